# Initial kernel scaffold; baseline (speedup 1.0000x reference)
import jax
import jax.numpy as jnp
from jax.experimental import pallas as pl


def kernel(x, edge_index, W1, b1, W2, b2, g1, be1, Wl2, bl2, g2, be2, Wl3, bl3, g3, be3, Wl4, bl4):
    raise NotImplementedError("write your pallas kernel here")


# speedup vs baseline: 9.2037x; 9.2037x over previous
import functools

import jax
import jax.numpy as jnp
from jax import lax
from jax.experimental import pallas as pl
from jax.experimental.pallas import tpu as pltpu
from jax.experimental.pallas import tpu_sc as plsc

_N = 100000
_NPAD = 100352
_RPT = _NPAD // 16
_RB = 512
_R = _NPAD // _RB
_CH = 128
_NP2 = 10
_EPS = 1e-5


def _mesh():
    return plsc.VectorSubcoreMesh(core_axis_name="c", subcore_axis_name="s")


def _bf(a):
    return a.astype(jnp.bfloat16).astype(jnp.float32)


def _mm(a, b):
    return lax.dot_general(a, b, (((1,), (0,)), ((), ())),
                           precision=lax.Precision.HIGHEST,
                           preferred_element_type=jnp.float32)


def _make_deg(chunks):
    per_tile = chunks // 32
    blk = 16
    blocks = per_tile // blk

    @functools.partial(
        pl.kernel,
        out_type=jax.ShapeDtypeStruct((2, 16, _RPT), jnp.float32),
        mesh=_mesh(),
        compiler_params=pltpu.CompilerParams(use_tc_tiling_on_sc=False),
        scratch_types=[
            pltpu.VMEM((blk, _CH), jnp.int32),
            pltpu.VMEM((_CH,), jnp.float32),
            pltpu.VMEM_SHARED((_NPAD,), jnp.float32),
        ],
    )
    def deg_kernel(dst_hbm, ones_hbm, z1_hbm, degp_hbm, dstbuf, ones_v, acc):
        cid = lax.axis_index("c")
        sid = lax.axis_index("s")
        wid = cid * 16 + sid
        pltpu.sync_copy(z1_hbm, acc.at[pl.ds(sid * _RPT, _RPT)])
        pltpu.sync_copy(ones_hbm, ones_v)
        plsc.subcore_barrier()
        base = wid * per_tile

        def blk_body(b, carry):
            pltpu.sync_copy(dst_hbm.at[pl.ds(base + b * blk, blk)], dstbuf)

            def ch_body(j, c2):
                pltpu.sync_copy(ones_v, acc.at[dstbuf.at[j]], add=True)
                return c2

            return lax.fori_loop(0, blk, ch_body, carry)

        lax.fori_loop(0, blocks, blk_body, 0)
        plsc.subcore_barrier()
        pltpu.sync_copy(acc.at[pl.ds(sid * _RPT, _RPT)], degp_hbm.at[cid, sid])

    return deg_kernel


def _make_conv1(chunks):
    per_tile = chunks // 32
    blk = 16
    blocks = per_tile // blk

    @functools.partial(
        pl.kernel,
        out_type=jax.ShapeDtypeStruct((2, 16, _RPT, 16), jnp.float32),
        mesh=_mesh(),
        compiler_params=pltpu.CompilerParams(use_tc_tiling_on_sc=False),
        scratch_types=[
            pltpu.VMEM((blk, _CH), jnp.int32),
            pltpu.VMEM((blk, _CH), jnp.int32),
            pltpu.VMEM((_CH, 16), jnp.float32),
            pltpu.VMEM_SHARED((_NPAD, 16), jnp.float32),
            pltpu.SemaphoreType.DMA,
        ],
    )
    def conv1_kernel(src_hbm, dst_hbm, y_hbm, zrows_hbm, outp_hbm,
                     srcbuf, dstbuf, rows, acc, sem):
        cid = lax.axis_index("c")
        sid = lax.axis_index("s")
        wid = cid * 16 + sid
        pltpu.sync_copy(zrows_hbm, acc.at[pl.ds(sid * _RPT, _RPT)])
        plsc.subcore_barrier()
        base = wid * per_tile

        def blk_body(b, carry):
            pltpu.sync_copy(src_hbm.at[pl.ds(base + b * blk, blk)], srcbuf)
            pltpu.sync_copy(dst_hbm.at[pl.ds(base + b * blk, blk)], dstbuf)

            def ch_body(j, c2):
                pltpu.async_copy(y_hbm.at[srcbuf.at[j]], rows, sem).wait()
                pltpu.sync_copy(rows, acc.at[dstbuf.at[j]], add=True)
                return c2

            return lax.fori_loop(0, blk, ch_body, carry)

        lax.fori_loop(0, blocks, blk_body, 0)
        plsc.subcore_barrier()
        pltpu.sync_copy(acc.at[pl.ds(sid * _RPT, _RPT)], outp_hbm.at[cid, sid])

    return conv1_kernel


def _make_conv2(chunks):
    per_tile = chunks // 16
    blk = 32
    blocks = per_tile // blk

    @functools.partial(
        pl.kernel,
        out_type=jax.ShapeDtypeStruct((_NP2, 16, _RPT, 16), jnp.float32),
        mesh=_mesh(),
        compiler_params=pltpu.CompilerParams(use_tc_tiling_on_sc=False),
        scratch_types=[
            pltpu.VMEM((blk, _CH), jnp.int32),
            pltpu.VMEM((blk, _CH), jnp.int32),
            pltpu.VMEM((_CH, 16), jnp.float32),
            pltpu.VMEM_SHARED((_NPAD, 16), jnp.float32),
            pltpu.SemaphoreType.DMA,
        ],
    )
    def conv2_kernel(src_hbm, dst_hbm, y3_hbm, zrows_hbm, out3_hbm,
                     srcbuf, dstbuf, rows, acc, sem):
        cid = lax.axis_index("c")
        sid = lax.axis_index("s")
        for q in range(_NP2 // 2):
            p = q * 2 + cid
            pltpu.sync_copy(zrows_hbm, acc.at[pl.ds(sid * _RPT, _RPT)])
            plsc.subcore_barrier()
            base = sid * per_tile

            def blk_body(b, carry):
                pltpu.sync_copy(src_hbm.at[pl.ds(base + b * blk, blk)], srcbuf)
                pltpu.sync_copy(dst_hbm.at[pl.ds(base + b * blk, blk)], dstbuf)

                def ch_body(j, c2):
                    pltpu.async_copy(y3_hbm.at[p].at[srcbuf.at[j]], rows,
                                     sem).wait()
                    pltpu.sync_copy(rows, acc.at[dstbuf.at[j]], add=True)
                    return c2

                return lax.fori_loop(0, blk, ch_body, carry)

            lax.fori_loop(0, blocks, blk_body, 0)
            plsc.subcore_barrier()
            pltpu.sync_copy(acc.at[pl.ds(sid * _RPT, _RPT)], out3_hbm.at[p, sid])
            plsc.subcore_barrier()

    return conv2_kernel


def _t1_call(degp_t, x16):
    def body(dg_ref, x_ref, y0_ref, dv_ref):
        d = dg_ref[:, 0:1] + dg_ref[:, 1:2] + 1.0
        dv = jnp.broadcast_to(lax.rsqrt(d), (_RB, 16))
        dv_ref[...] = dv
        y0_ref[...] = _bf(x_ref[...]) * dv

    return pl.pallas_call(
        body,
        grid=(_R,),
        in_specs=[
            pl.BlockSpec((_RB, 2), lambda r: (r, 0)),
            pl.BlockSpec((_RB, 16), lambda r: (r, 0)),
        ],
        out_specs=[
            pl.BlockSpec((_RB, 16), lambda r: (r, 0)),
            pl.BlockSpec((_RB, 16), lambda r: (r, 0)),
        ],
        out_shape=[
            jax.ShapeDtypeStruct((_NPAD, 16), jnp.float32),
            jax.ShapeDtypeStruct((_NPAD, 16), jnp.float32),
        ],
    )(degp_t, x16)


def _t2_call(out1p, y0, dv16, w1p, b1p):
    def body(op_ref, y0_ref, dv_ref, w_ref, b_ref, y1_ref):
        dv = dv_ref[...]
        agg = (op_ref[0] + op_ref[1] + y0_ref[...]) * dv
        h = jnp.maximum(_mm(agg, _bf(w_ref[...])) + b_ref[...], 0.0)
        for p in range(_NP2):
            y1_ref[p] = _bf(h[:, p * 16:(p + 1) * 16]) * dv

    return pl.pallas_call(
        body,
        grid=(_R,),
        in_specs=[
            pl.BlockSpec((2, _RB, 16), lambda r: (0, r, 0)),
            pl.BlockSpec((_RB, 16), lambda r: (r, 0)),
            pl.BlockSpec((_RB, 16), lambda r: (r, 0)),
            pl.BlockSpec((16, 160), lambda r: (0, 0)),
            pl.BlockSpec((1, 160), lambda r: (0, 0)),
        ],
        out_specs=pl.BlockSpec((_NP2, _RB, 16), lambda r: (0, r, 0)),
        out_shape=jax.ShapeDtypeStruct((_NP2, _NPAD, 16), jnp.float32),
    )(out1p, y0, dv16, w1p, b1p)


def _d1_call(out3, y1, dv16, w2p, b2r):
    def body(o3_ref, y1_ref, dv_ref, w_ref, b_ref, h_ref, s_ref):
        r = pl.program_id(0)
        dv = dv_ref[...]
        ms = [(o3_ref[p] + y1_ref[p]) * dv for p in range(_NP2)]
        m = jnp.concatenate(ms, axis=1)
        h = _mm(m, _bf(w_ref[...])) + b_ref[...]
        h_ref[...] = h
        mask = (lax.broadcasted_iota(jnp.int32, (_RB, 1), 0) + r * _RB) < _N
        hm = jnp.where(mask, h, 0.0)

        @pl.when(r == 0)
        def _():
            s_ref[...] = jnp.zeros((8, 200), jnp.float32)

        s_ref[0:1, :] = s_ref[0:1, :] + jnp.sum(hm, axis=0, keepdims=True)
        s_ref[1:2, :] = s_ref[1:2, :] + jnp.sum(hm * hm, axis=0, keepdims=True)

    return pl.pallas_call(
        body,
        grid=(_R,),
        in_specs=[
            pl.BlockSpec((_NP2, _RB, 16), lambda r: (0, r, 0)),
            pl.BlockSpec((_NP2, _RB, 16), lambda r: (0, r, 0)),
            pl.BlockSpec((_RB, 16), lambda r: (r, 0)),
            pl.BlockSpec((160, 200), lambda r: (0, 0)),
            pl.BlockSpec((1, 200), lambda r: (0, 0)),
        ],
        out_specs=[
            pl.BlockSpec((_RB, 200), lambda r: (r, 0)),
            pl.BlockSpec((8, 200), lambda r: (0, 0)),
        ],
        out_shape=[
            jax.ShapeDtypeStruct((_NPAD, 200), jnp.float32),
            jax.ShapeDtypeStruct((8, 200), jnp.float32),
        ],
    )(out3, y1, dv16, w2p, b2r)


def _bn_mlp_call(h, s, g, be, w, b, fin, fout, last):
    def body(h_ref, s_ref, g_ref, be_ref, w_ref, b_ref, o_ref, so_ref):
        r = pl.program_id(0)
        mean = s_ref[0:1, :] * (1.0 / _N)
        var = s_ref[1:2, :] * (1.0 / _N) - mean * mean
        scale = g_ref[...] * lax.rsqrt(var + _EPS)
        z = jnp.maximum((h_ref[...] - mean) * scale + be_ref[...], 0.0)
        o = lax.dot_general(
            z.astype(jnp.bfloat16), w_ref[...].astype(jnp.bfloat16),
            (((1,), (0,)), ((), ())),
            preferred_element_type=jnp.float32) + b_ref[...]
        o_ref[...] = o
        if not last:
            mask = (lax.broadcasted_iota(jnp.int32, (_RB, 1), 0)
                    + r * _RB) < _N
            om = jnp.where(mask, o, 0.0)

            @pl.when(r == 0)
            def _():
                so_ref[...] = jnp.zeros((8, fout), jnp.float32)

            so_ref[0:1, :] = so_ref[0:1, :] + jnp.sum(om, axis=0, keepdims=True)
            so_ref[1:2, :] = so_ref[1:2, :] + jnp.sum(om * om, axis=0,
                                                      keepdims=True)

    out_specs = [pl.BlockSpec((_RB, fout), lambda r: (r, 0)),
                 pl.BlockSpec((8, fout), lambda r: (0, 0))]
    out_shape = [jax.ShapeDtypeStruct((_NPAD, fout), jnp.float32),
                 jax.ShapeDtypeStruct((8, fout), jnp.float32)]
    res = pl.pallas_call(
        body,
        grid=(_R,),
        in_specs=[
            pl.BlockSpec((_RB, fin), lambda r: (r, 0)),
            pl.BlockSpec((8, fin), lambda r: (0, 0)),
            pl.BlockSpec((1, fin), lambda r: (0, 0)),
            pl.BlockSpec((1, fin), lambda r: (0, 0)),
            pl.BlockSpec((fin, fout), lambda r: (0, 0)),
            pl.BlockSpec((1, fout), lambda r: (0, 0)),
        ],
        out_specs=out_specs,
        out_shape=out_shape,
    )(h, s, g, be, w, b)
    return res[0], res[1]


def kernel(x, edge_index, W1, b1, W2, b2, g1, be1, Wl2, bl2, g2, be2,
           Wl3, bl3, g3, be3, Wl4, bl4):
    n = x.shape[0]
    e = edge_index.shape[1]
    blocks1 = -(-e // (_CH * 32 * 16))
    chunks = blocks1 * 32 * 16
    epad = chunks * _CH

    src = edge_index[0].astype(jnp.int32)
    dst = edge_index[1].astype(jnp.int32)
    fill = jnp.full((epad - e,), _NPAD - 1, jnp.int32)
    srcp = jnp.concatenate([src, fill]).reshape(chunks, _CH)
    dstp = jnp.concatenate([dst, fill]).reshape(chunks, _CH)

    x16 = jnp.pad(x, ((0, _NPAD - n), (0, 16 - x.shape[1])))
    w1p = jnp.pad(W1, ((0, 16 - W1.shape[0]), (0, 160 - W1.shape[1])))
    b1p = jnp.pad(b1, (0, 160 - b1.shape[0])).reshape(1, 160)
    w2p = jnp.pad(W2, ((0, 160 - W2.shape[0]), (0, 0)))
    b2r = b2.reshape(1, 200)
    wl4p = jnp.pad(Wl4, ((0, 0), (0, 8 - Wl4.shape[1])))
    bl4p = jnp.pad(bl4, (0, 8 - bl4.shape[0])).reshape(1, 8)

    ones = jnp.ones((_CH,), jnp.float32)
    z1 = jnp.zeros((_RPT,), jnp.float32)
    zrows = jnp.zeros((_RPT, 16), jnp.float32)

    degp = _make_deg(chunks)(dstp, ones, z1)
    degp_t = degp.reshape(2, _NPAD).T
    y0, dv16 = _t1_call(degp_t, x16)

    out1p = _make_conv1(chunks)(srcp, dstp, y0, zrows)
    out1p = out1p.reshape(2, _NPAD, 16)
    y1 = _t2_call(out1p, y0, dv16, w1p, b1p)

    out3 = _make_conv2(chunks)(srcp, dstp, y1, zrows)
    out3 = out3.reshape(_NP2, _NPAD, 16)
    h2, s1 = _d1_call(out3, y1, dv16, w2p, b2r)

    h3, s2 = _bn_mlp_call(h2, s1, g1.reshape(1, 200), be1.reshape(1, 200),
                          Wl2, bl2.reshape(1, 400), 200, 400, False)
    h4, s3 = _bn_mlp_call(h3, s2, g2.reshape(1, 400), be2.reshape(1, 400),
                          Wl3, bl3.reshape(1, 200), 400, 200, False)
    h5, _ = _bn_mlp_call(h4, s3, g3.reshape(1, 200), be3.reshape(1, 200),
                         wl4p, bl4p, 200, 8, True)
    return h5[:n, 0:1]


# speedup vs baseline: 14.7294x; 1.6004x over previous
import functools

import jax
import jax.numpy as jnp
from jax import lax
from jax.experimental import pallas as pl
from jax.experimental.pallas import tpu as pltpu
from jax.experimental.pallas import tpu_sc as plsc

_N = 100000
_NPAD = 100352
_RPT = _NPAD // 16
_RB = 512
_R = _NPAD // _RB
_CH = 128
_EB = 1024
_NP2 = 10
_EPS = 1e-5


def _mesh():
    return plsc.VectorSubcoreMesh(core_axis_name="c", subcore_axis_name="s")


def _bf(a):
    return a.astype(jnp.bfloat16).astype(jnp.float32)


def _mm(a, b):
    return lax.dot_general(a, b, (((1,), (0,)), ((), ())),
                           precision=lax.Precision.HIGHEST,
                           preferred_element_type=jnp.float32)


def _make_deg(epad):
    per_tile = epad // 32
    blocks = per_tile // _EB

    @functools.partial(
        pl.kernel,
        out_type=jax.ShapeDtypeStruct((2, 16, _RPT), jnp.float32),
        mesh=_mesh(),
        compiler_params=pltpu.CompilerParams(use_tc_tiling_on_sc=False),
        scratch_types=[
            pltpu.VMEM((_EB,), jnp.int32),
            pltpu.VMEM((_EB,), jnp.float32),
            pltpu.VMEM_SHARED((_NPAD,), jnp.float32),
        ],
    )
    def deg_kernel(dst_hbm, ones_hbm, z1_hbm, degp_hbm, dstbuf, ones_v, acc):
        cid = lax.axis_index("c")
        sid = lax.axis_index("s")
        wid = cid * 16 + sid
        pltpu.sync_copy(z1_hbm, acc.at[pl.ds(sid * _RPT, _RPT)])
        pltpu.sync_copy(ones_hbm, ones_v)
        plsc.subcore_barrier()
        base = wid * per_tile

        def blk_body(b, carry):
            pltpu.sync_copy(dst_hbm.at[pl.ds(base + b * _EB, _EB)], dstbuf)
            pltpu.sync_copy(ones_v, acc.at[dstbuf], add=True)
            return carry

        lax.fori_loop(0, blocks, blk_body, 0)
        plsc.subcore_barrier()
        pltpu.sync_copy(acc.at[pl.ds(sid * _RPT, _RPT)], degp_hbm.at[cid, sid])

    return deg_kernel


def _make_conv1(epad):
    per_tile = epad // 32
    blocks = per_tile // _EB

    @functools.partial(
        pl.kernel,
        out_type=jax.ShapeDtypeStruct((2, 16, _RPT, 16), jnp.float32),
        mesh=_mesh(),
        compiler_params=pltpu.CompilerParams(use_tc_tiling_on_sc=False),
        scratch_types=[
            pltpu.VMEM((_EB,), jnp.int32),
            pltpu.VMEM((_EB,), jnp.int32),
            pltpu.VMEM((_EB, 16), jnp.float32),
            pltpu.VMEM_SHARED((_NPAD, 16), jnp.float32),
            pltpu.SemaphoreType.DMA,
        ],
    )
    def conv1_kernel(src_hbm, dst_hbm, y_hbm, zrows_hbm, outp_hbm,
                     srcbuf, dstbuf, rows, acc, sem):
        cid = lax.axis_index("c")
        sid = lax.axis_index("s")
        wid = cid * 16 + sid
        pltpu.sync_copy(zrows_hbm, acc.at[pl.ds(sid * _RPT, _RPT)])
        plsc.subcore_barrier()
        base = wid * per_tile

        def blk_body(b, carry):
            pltpu.sync_copy(src_hbm.at[pl.ds(base + b * _EB, _EB)], srcbuf)
            pltpu.sync_copy(dst_hbm.at[pl.ds(base + b * _EB, _EB)], dstbuf)
            pltpu.async_copy(y_hbm.at[srcbuf], rows, sem).wait()
            pltpu.sync_copy(rows, acc.at[dstbuf], add=True)
            return carry

        lax.fori_loop(0, blocks, blk_body, 0)
        plsc.subcore_barrier()
        pltpu.sync_copy(acc.at[pl.ds(sid * _RPT, _RPT)], outp_hbm.at[cid, sid])

    return conv1_kernel


def _make_conv2(epad):
    per_tile = epad // 16
    blocks = per_tile // _EB

    @functools.partial(
        pl.kernel,
        out_type=jax.ShapeDtypeStruct((_NP2, 16, _RPT, 16), jnp.float32),
        mesh=_mesh(),
        compiler_params=pltpu.CompilerParams(use_tc_tiling_on_sc=False),
        scratch_types=[
            pltpu.VMEM((_EB,), jnp.int32),
            pltpu.VMEM((_EB,), jnp.int32),
            pltpu.VMEM((_EB, 16), jnp.float32),
            pltpu.VMEM_SHARED((_NPAD, 16), jnp.float32),
            pltpu.SemaphoreType.DMA,
        ],
    )
    def conv2_kernel(src_hbm, dst_hbm, y3_hbm, zrows_hbm, out3_hbm,
                     srcbuf, dstbuf, rows, acc, sem):
        cid = lax.axis_index("c")
        sid = lax.axis_index("s")
        for q in range(_NP2 // 2):
            p = q * 2 + cid
            pltpu.sync_copy(zrows_hbm, acc.at[pl.ds(sid * _RPT, _RPT)])
            plsc.subcore_barrier()
            base = sid * per_tile

            def blk_body(b, carry):
                pltpu.sync_copy(src_hbm.at[pl.ds(base + b * _EB, _EB)], srcbuf)
                pltpu.sync_copy(dst_hbm.at[pl.ds(base + b * _EB, _EB)], dstbuf)
                pltpu.async_copy(y3_hbm.at[p].at[srcbuf], rows, sem).wait()
                pltpu.sync_copy(rows, acc.at[dstbuf], add=True)
                return carry

            lax.fori_loop(0, blocks, blk_body, 0)
            plsc.subcore_barrier()
            pltpu.sync_copy(acc.at[pl.ds(sid * _RPT, _RPT)], out3_hbm.at[p, sid])
            plsc.subcore_barrier()

    return conv2_kernel


def _t1_call(degp_t, x16):
    def body(dg_ref, x_ref, y0_ref, dv_ref):
        d = dg_ref[:, 0:1] + dg_ref[:, 1:2] + 1.0
        dv = jnp.broadcast_to(lax.rsqrt(d), (_RB, 16))
        dv_ref[...] = dv
        y0_ref[...] = _bf(x_ref[...]) * dv

    return pl.pallas_call(
        body,
        grid=(_R,),
        in_specs=[
            pl.BlockSpec((_RB, 2), lambda r: (r, 0)),
            pl.BlockSpec((_RB, 16), lambda r: (r, 0)),
        ],
        out_specs=[
            pl.BlockSpec((_RB, 16), lambda r: (r, 0)),
            pl.BlockSpec((_RB, 16), lambda r: (r, 0)),
        ],
        out_shape=[
            jax.ShapeDtypeStruct((_NPAD, 16), jnp.float32),
            jax.ShapeDtypeStruct((_NPAD, 16), jnp.float32),
        ],
    )(degp_t, x16)


def _t2_call(out1p, y0, dv16, w1p, b1p):
    def body(op_ref, y0_ref, dv_ref, w_ref, b_ref, y1_ref):
        dv = dv_ref[...]
        agg = (op_ref[0] + op_ref[1] + y0_ref[...]) * dv
        h = jnp.maximum(_mm(agg, _bf(w_ref[...])) + b_ref[...], 0.0)
        for p in range(_NP2):
            y1_ref[p] = _bf(h[:, p * 16:(p + 1) * 16]) * dv

    return pl.pallas_call(
        body,
        grid=(_R,),
        in_specs=[
            pl.BlockSpec((2, _RB, 16), lambda r: (0, r, 0)),
            pl.BlockSpec((_RB, 16), lambda r: (r, 0)),
            pl.BlockSpec((_RB, 16), lambda r: (r, 0)),
            pl.BlockSpec((16, 160), lambda r: (0, 0)),
            pl.BlockSpec((1, 160), lambda r: (0, 0)),
        ],
        out_specs=pl.BlockSpec((_NP2, _RB, 16), lambda r: (0, r, 0)),
        out_shape=jax.ShapeDtypeStruct((_NP2, _NPAD, 16), jnp.float32),
    )(out1p, y0, dv16, w1p, b1p)


def _d1_call(out3, y1, dv16, w2p, b2r):
    def body(o3_ref, y1_ref, dv_ref, w_ref, b_ref, h_ref, s_ref):
        r = pl.program_id(0)
        dv = dv_ref[...]
        ms = [(o3_ref[p] + y1_ref[p]) * dv for p in range(_NP2)]
        m = jnp.concatenate(ms, axis=1)
        h = _mm(m, _bf(w_ref[...])) + b_ref[...]
        h_ref[...] = h
        mask = (lax.broadcasted_iota(jnp.int32, (_RB, 1), 0) + r * _RB) < _N
        hm = jnp.where(mask, h, 0.0)

        @pl.when(r == 0)
        def _():
            s_ref[...] = jnp.zeros((8, 200), jnp.float32)

        s_ref[0:1, :] = s_ref[0:1, :] + jnp.sum(hm, axis=0, keepdims=True)
        s_ref[1:2, :] = s_ref[1:2, :] + jnp.sum(hm * hm, axis=0, keepdims=True)

    return pl.pallas_call(
        body,
        grid=(_R,),
        in_specs=[
            pl.BlockSpec((_NP2, _RB, 16), lambda r: (0, r, 0)),
            pl.BlockSpec((_NP2, _RB, 16), lambda r: (0, r, 0)),
            pl.BlockSpec((_RB, 16), lambda r: (r, 0)),
            pl.BlockSpec((160, 200), lambda r: (0, 0)),
            pl.BlockSpec((1, 200), lambda r: (0, 0)),
        ],
        out_specs=[
            pl.BlockSpec((_RB, 200), lambda r: (r, 0)),
            pl.BlockSpec((8, 200), lambda r: (0, 0)),
        ],
        out_shape=[
            jax.ShapeDtypeStruct((_NPAD, 200), jnp.float32),
            jax.ShapeDtypeStruct((8, 200), jnp.float32),
        ],
    )(out3, y1, dv16, w2p, b2r)


def _bn_mlp_call(h, s, g, be, w, b, fin, fout, last):
    def body(h_ref, s_ref, g_ref, be_ref, w_ref, b_ref, o_ref, so_ref):
        r = pl.program_id(0)
        mean = s_ref[0:1, :] * (1.0 / _N)
        var = s_ref[1:2, :] * (1.0 / _N) - mean * mean
        scale = g_ref[...] * lax.rsqrt(var + _EPS)
        z = jnp.maximum((h_ref[...] - mean) * scale + be_ref[...], 0.0)
        o = lax.dot_general(
            z.astype(jnp.bfloat16), w_ref[...].astype(jnp.bfloat16),
            (((1,), (0,)), ((), ())),
            preferred_element_type=jnp.float32) + b_ref[...]
        o_ref[...] = o
        if not last:
            mask = (lax.broadcasted_iota(jnp.int32, (_RB, 1), 0)
                    + r * _RB) < _N
            om = jnp.where(mask, o, 0.0)

            @pl.when(r == 0)
            def _():
                so_ref[...] = jnp.zeros((8, fout), jnp.float32)

            so_ref[0:1, :] = so_ref[0:1, :] + jnp.sum(om, axis=0, keepdims=True)
            so_ref[1:2, :] = so_ref[1:2, :] + jnp.sum(om * om, axis=0,
                                                      keepdims=True)

    out_specs = [pl.BlockSpec((_RB, fout), lambda r: (r, 0)),
                 pl.BlockSpec((8, fout), lambda r: (0, 0))]
    out_shape = [jax.ShapeDtypeStruct((_NPAD, fout), jnp.float32),
                 jax.ShapeDtypeStruct((8, fout), jnp.float32)]
    res = pl.pallas_call(
        body,
        grid=(_R,),
        in_specs=[
            pl.BlockSpec((_RB, fin), lambda r: (r, 0)),
            pl.BlockSpec((8, fin), lambda r: (0, 0)),
            pl.BlockSpec((1, fin), lambda r: (0, 0)),
            pl.BlockSpec((1, fin), lambda r: (0, 0)),
            pl.BlockSpec((fin, fout), lambda r: (0, 0)),
            pl.BlockSpec((1, fout), lambda r: (0, 0)),
        ],
        out_specs=out_specs,
        out_shape=out_shape,
    )(h, s, g, be, w, b)
    return res[0], res[1]


def kernel(x, edge_index, W1, b1, W2, b2, g1, be1, Wl2, bl2, g2, be2,
           Wl3, bl3, g3, be3, Wl4, bl4):
    n = x.shape[0]
    e = edge_index.shape[1]
    blocks1 = -(-e // (_CH * 32 * 16))
    chunks = blocks1 * 32 * 16
    epad = chunks * _CH

    src = edge_index[0].astype(jnp.int32)
    dst = edge_index[1].astype(jnp.int32)
    fill = jnp.full((epad - e,), _NPAD - 1, jnp.int32)
    srcp = jnp.concatenate([src, fill])
    dstp = jnp.concatenate([dst, fill])

    x16 = jnp.pad(x, ((0, _NPAD - n), (0, 16 - x.shape[1])))
    w1p = jnp.pad(W1, ((0, 16 - W1.shape[0]), (0, 160 - W1.shape[1])))
    b1p = jnp.pad(b1, (0, 160 - b1.shape[0])).reshape(1, 160)
    w2p = jnp.pad(W2, ((0, 160 - W2.shape[0]), (0, 0)))
    b2r = b2.reshape(1, 200)
    wl4p = jnp.pad(Wl4, ((0, 0), (0, 8 - Wl4.shape[1])))
    bl4p = jnp.pad(bl4, (0, 8 - bl4.shape[0])).reshape(1, 8)

    ones = jnp.ones((_EB,), jnp.float32)
    z1 = jnp.zeros((_RPT,), jnp.float32)
    zrows = jnp.zeros((_RPT, 16), jnp.float32)

    degp = _make_deg(epad)(dstp, ones, z1)
    degp_t = degp.reshape(2, _NPAD).T
    y0, dv16 = _t1_call(degp_t, x16)

    out1p = _make_conv1(epad)(srcp, dstp, y0, zrows)
    out1p = out1p.reshape(2, _NPAD, 16)
    y1 = _t2_call(out1p, y0, dv16, w1p, b1p)

    out3 = _make_conv2(epad)(srcp, dstp, y1, zrows)
    out3 = out3.reshape(_NP2, _NPAD, 16)
    h2, s1 = _d1_call(out3, y1, dv16, w2p, b2r)

    h3, s2 = _bn_mlp_call(h2, s1, g1.reshape(1, 200), be1.reshape(1, 200),
                          Wl2, bl2.reshape(1, 400), 200, 400, False)
    h4, s3 = _bn_mlp_call(h3, s2, g2.reshape(1, 400), be2.reshape(1, 400),
                          Wl3, bl3.reshape(1, 200), 400, 200, False)
    h5, _ = _bn_mlp_call(h4, s3, g3.reshape(1, 200), be3.reshape(1, 200),
                         wl4p, bl4p, 200, 8, True)
    return h5[:n, 0:1]


# speedup vs baseline: 16.8023x; 1.1407x over previous
import functools

import jax
import jax.numpy as jnp
from jax import lax
from jax.experimental import pallas as pl
from jax.experimental.pallas import tpu as pltpu
from jax.experimental.pallas import tpu_sc as plsc

_N = 100000
_NPAD = 100352
_RPT = _NPAD // 16
_RB = 512
_R = _NPAD // _RB
_CH = 128
_EB = 512
_NP2 = 10
_EPS = 1e-5


def _mesh():
    return plsc.VectorSubcoreMesh(core_axis_name="c", subcore_axis_name="s")


def _bf(a):
    return a.astype(jnp.bfloat16).astype(jnp.float32)


def _mm(a, b):
    return lax.dot_general(a, b, (((1,), (0,)), ((), ())),
                           precision=lax.Precision.HIGHEST,
                           preferred_element_type=jnp.float32)


def _make_deg(epad):
    per_tile = epad // 32
    blocks = per_tile // _EB

    @functools.partial(
        pl.kernel,
        out_type=jax.ShapeDtypeStruct((2, 16, _RPT), jnp.float32),
        mesh=_mesh(),
        compiler_params=pltpu.CompilerParams(use_tc_tiling_on_sc=False),
        scratch_types=[
            pltpu.VMEM((_EB,), jnp.int32),
            pltpu.VMEM((_EB,), jnp.float32),
            pltpu.VMEM_SHARED((_NPAD,), jnp.float32),
        ],
    )
    def deg_kernel(dst_hbm, ones_hbm, z1_hbm, degp_hbm, dstbuf, ones_v, acc):
        cid = lax.axis_index("c")
        sid = lax.axis_index("s")
        wid = cid * 16 + sid
        pltpu.sync_copy(z1_hbm, acc.at[pl.ds(sid * _RPT, _RPT)])
        pltpu.sync_copy(ones_hbm, ones_v)
        plsc.subcore_barrier()
        base = wid * per_tile

        def blk_body(b, carry):
            pltpu.sync_copy(dst_hbm.at[pl.ds(base + b * _EB, _EB)], dstbuf)
            pltpu.sync_copy(ones_v, acc.at[dstbuf], add=True)
            return carry

        lax.fori_loop(0, blocks, blk_body, 0)
        plsc.subcore_barrier()
        pltpu.sync_copy(acc.at[pl.ds(sid * _RPT, _RPT)], degp_hbm.at[cid, sid])

    return deg_kernel


def _make_conv1(epad):
    per_tile = epad // 32
    blocks = per_tile // _EB

    @functools.partial(
        pl.kernel,
        out_type=jax.ShapeDtypeStruct((2, 16, _RPT, 16), jnp.float32),
        mesh=_mesh(),
        compiler_params=pltpu.CompilerParams(use_tc_tiling_on_sc=False),
        scratch_types=[
            pltpu.VMEM((_EB,), jnp.int32),
            pltpu.VMEM((_EB,), jnp.int32),
            pltpu.VMEM((_EB,), jnp.int32),
            pltpu.VMEM((_EB,), jnp.int32),
            pltpu.VMEM((_EB, 16), jnp.float32),
            pltpu.VMEM((_EB, 16), jnp.float32),
            pltpu.VMEM_SHARED((_NPAD, 16), jnp.float32),
            pltpu.SemaphoreType.DMA,
            pltpu.SemaphoreType.DMA,
        ],
    )
    def conv1_kernel(src_hbm, dst_hbm, y_hbm, zrows_hbm, outp_hbm,
                     srcbuf0, dstbuf0, srcbuf1, dstbuf1, rows0, rows1,
                     acc, sem0, sem1):
        cid = lax.axis_index("c")
        sid = lax.axis_index("s")
        wid = cid * 16 + sid
        pltpu.sync_copy(zrows_hbm, acc.at[pl.ds(sid * _RPT, _RPT)])
        plsc.subcore_barrier()
        base = wid * per_tile
        pairs = blocks // 2

        pltpu.sync_copy(src_hbm.at[pl.ds(base, _EB)], srcbuf0)
        pltpu.sync_copy(dst_hbm.at[pl.ds(base, _EB)], dstbuf0)
        pltpu.async_copy(y_hbm.at[srcbuf0], rows0, sem0)

        def pair_body(t, carry):
            e1 = base + (2 * t + 1) * _EB
            pltpu.sync_copy(src_hbm.at[pl.ds(e1, _EB)], srcbuf1)
            pltpu.sync_copy(dst_hbm.at[pl.ds(e1, _EB)], dstbuf1)
            pltpu.async_copy(y_hbm.at[srcbuf1], rows1, sem1)
            pltpu.make_async_copy(y_hbm.at[srcbuf0], rows0, sem0).wait()
            pltpu.sync_copy(rows0, acc.at[dstbuf0], add=True)

            @pl.when(t + 1 < pairs)
            def _():
                e2 = base + (2 * t + 2) * _EB
                pltpu.sync_copy(src_hbm.at[pl.ds(e2, _EB)], srcbuf0)
                pltpu.sync_copy(dst_hbm.at[pl.ds(e2, _EB)], dstbuf0)
                pltpu.async_copy(y_hbm.at[srcbuf0], rows0, sem0)

            pltpu.make_async_copy(y_hbm.at[srcbuf1], rows1, sem1).wait()
            pltpu.sync_copy(rows1, acc.at[dstbuf1], add=True)
            return carry

        lax.fori_loop(0, pairs, pair_body, 0)
        plsc.subcore_barrier()
        pltpu.sync_copy(acc.at[pl.ds(sid * _RPT, _RPT)], outp_hbm.at[cid, sid])

    return conv1_kernel


def _make_conv2(epad):
    per_tile = epad // 16
    blocks = per_tile // _EB

    @functools.partial(
        pl.kernel,
        out_type=jax.ShapeDtypeStruct((_NP2, 16, _RPT, 16), jnp.float32),
        mesh=_mesh(),
        compiler_params=pltpu.CompilerParams(use_tc_tiling_on_sc=False),
        scratch_types=[
            pltpu.VMEM((_EB,), jnp.int32),
            pltpu.VMEM((_EB,), jnp.int32),
            pltpu.VMEM((_EB,), jnp.int32),
            pltpu.VMEM((_EB,), jnp.int32),
            pltpu.VMEM((_EB, 16), jnp.float32),
            pltpu.VMEM((_EB, 16), jnp.float32),
            pltpu.VMEM_SHARED((_NPAD, 16), jnp.float32),
            pltpu.SemaphoreType.DMA,
            pltpu.SemaphoreType.DMA,
        ],
    )
    def conv2_kernel(src_hbm, dst_hbm, y3_hbm, zrows_hbm, out3_hbm,
                     srcbuf0, dstbuf0, srcbuf1, dstbuf1, rows0, rows1,
                     acc, sem0, sem1):
        cid = lax.axis_index("c")
        sid = lax.axis_index("s")
        for q in range(_NP2 // 2):
            p = q * 2 + cid
            yp = y3_hbm.at[p]
            pltpu.sync_copy(zrows_hbm, acc.at[pl.ds(sid * _RPT, _RPT)])
            plsc.subcore_barrier()
            base = sid * per_tile
            pairs = blocks // 2

            pltpu.sync_copy(src_hbm.at[pl.ds(base, _EB)], srcbuf0)
            pltpu.sync_copy(dst_hbm.at[pl.ds(base, _EB)], dstbuf0)
            pltpu.async_copy(yp.at[srcbuf0], rows0, sem0)

            def pair_body(t, carry):
                e1 = base + (2 * t + 1) * _EB
                pltpu.sync_copy(src_hbm.at[pl.ds(e1, _EB)], srcbuf1)
                pltpu.sync_copy(dst_hbm.at[pl.ds(e1, _EB)], dstbuf1)
                pltpu.async_copy(yp.at[srcbuf1], rows1, sem1)
                pltpu.make_async_copy(yp.at[srcbuf0], rows0, sem0).wait()
                pltpu.sync_copy(rows0, acc.at[dstbuf0], add=True)

                @pl.when(t + 1 < pairs)
                def _():
                    e2 = base + (2 * t + 2) * _EB
                    pltpu.sync_copy(src_hbm.at[pl.ds(e2, _EB)], srcbuf0)
                    pltpu.sync_copy(dst_hbm.at[pl.ds(e2, _EB)], dstbuf0)
                    pltpu.async_copy(yp.at[srcbuf0], rows0, sem0)

                pltpu.make_async_copy(yp.at[srcbuf1], rows1, sem1).wait()
                pltpu.sync_copy(rows1, acc.at[dstbuf1], add=True)
                return carry

            lax.fori_loop(0, pairs, pair_body, 0)
            plsc.subcore_barrier()
            pltpu.sync_copy(acc.at[pl.ds(sid * _RPT, _RPT)], out3_hbm.at[p, sid])
            plsc.subcore_barrier()

    return conv2_kernel


def _t1_call(degp_t, x16):
    def body(dg_ref, x_ref, y0_ref, dv_ref):
        d = dg_ref[:, 0:1] + dg_ref[:, 1:2] + 1.0
        dv = jnp.broadcast_to(lax.rsqrt(d), (_RB, 16))
        dv_ref[...] = dv
        y0_ref[...] = _bf(x_ref[...]) * dv

    return pl.pallas_call(
        body,
        grid=(_R,),
        in_specs=[
            pl.BlockSpec((_RB, 2), lambda r: (r, 0)),
            pl.BlockSpec((_RB, 16), lambda r: (r, 0)),
        ],
        out_specs=[
            pl.BlockSpec((_RB, 16), lambda r: (r, 0)),
            pl.BlockSpec((_RB, 16), lambda r: (r, 0)),
        ],
        out_shape=[
            jax.ShapeDtypeStruct((_NPAD, 16), jnp.float32),
            jax.ShapeDtypeStruct((_NPAD, 16), jnp.float32),
        ],
    )(degp_t, x16)


def _t2_call(out1p, y0, dv16, w1p, b1p):
    def body(op_ref, y0_ref, dv_ref, w_ref, b_ref, y1_ref):
        dv = dv_ref[...]
        agg = (op_ref[0] + op_ref[1] + y0_ref[...]) * dv
        h = jnp.maximum(_mm(agg, _bf(w_ref[...])) + b_ref[...], 0.0)
        for p in range(_NP2):
            y1_ref[p] = _bf(h[:, p * 16:(p + 1) * 16]) * dv

    return pl.pallas_call(
        body,
        grid=(_R,),
        in_specs=[
            pl.BlockSpec((2, _RB, 16), lambda r: (0, r, 0)),
            pl.BlockSpec((_RB, 16), lambda r: (r, 0)),
            pl.BlockSpec((_RB, 16), lambda r: (r, 0)),
            pl.BlockSpec((16, 160), lambda r: (0, 0)),
            pl.BlockSpec((1, 160), lambda r: (0, 0)),
        ],
        out_specs=pl.BlockSpec((_NP2, _RB, 16), lambda r: (0, r, 0)),
        out_shape=jax.ShapeDtypeStruct((_NP2, _NPAD, 16), jnp.float32),
    )(out1p, y0, dv16, w1p, b1p)


def _d1_call(out3, y1, dv16, w2p, b2r):
    def body(o3_ref, y1_ref, dv_ref, w_ref, b_ref, h_ref, s_ref):
        r = pl.program_id(0)
        dv = dv_ref[...]
        ms = [(o3_ref[p] + y1_ref[p]) * dv for p in range(_NP2)]
        m = jnp.concatenate(ms, axis=1)
        h = _mm(m, _bf(w_ref[...])) + b_ref[...]
        h_ref[...] = h
        mask = (lax.broadcasted_iota(jnp.int32, (_RB, 1), 0) + r * _RB) < _N
        hm = jnp.where(mask, h, 0.0)

        @pl.when(r == 0)
        def _():
            s_ref[...] = jnp.zeros((8, 200), jnp.float32)

        s_ref[0:1, :] = s_ref[0:1, :] + jnp.sum(hm, axis=0, keepdims=True)
        s_ref[1:2, :] = s_ref[1:2, :] + jnp.sum(hm * hm, axis=0, keepdims=True)

    return pl.pallas_call(
        body,
        grid=(_R,),
        in_specs=[
            pl.BlockSpec((_NP2, _RB, 16), lambda r: (0, r, 0)),
            pl.BlockSpec((_NP2, _RB, 16), lambda r: (0, r, 0)),
            pl.BlockSpec((_RB, 16), lambda r: (r, 0)),
            pl.BlockSpec((160, 200), lambda r: (0, 0)),
            pl.BlockSpec((1, 200), lambda r: (0, 0)),
        ],
        out_specs=[
            pl.BlockSpec((_RB, 200), lambda r: (r, 0)),
            pl.BlockSpec((8, 200), lambda r: (0, 0)),
        ],
        out_shape=[
            jax.ShapeDtypeStruct((_NPAD, 200), jnp.float32),
            jax.ShapeDtypeStruct((8, 200), jnp.float32),
        ],
    )(out3, y1, dv16, w2p, b2r)


def _bn_mlp_call(h, s, g, be, w, b, fin, fout, last):
    def body(h_ref, s_ref, g_ref, be_ref, w_ref, b_ref, o_ref, so_ref):
        r = pl.program_id(0)
        mean = s_ref[0:1, :] * (1.0 / _N)
        var = s_ref[1:2, :] * (1.0 / _N) - mean * mean
        scale = g_ref[...] * lax.rsqrt(var + _EPS)
        z = jnp.maximum((h_ref[...] - mean) * scale + be_ref[...], 0.0)
        o = lax.dot_general(
            z.astype(jnp.bfloat16), w_ref[...].astype(jnp.bfloat16),
            (((1,), (0,)), ((), ())),
            preferred_element_type=jnp.float32) + b_ref[...]
        o_ref[...] = o
        if not last:
            mask = (lax.broadcasted_iota(jnp.int32, (_RB, 1), 0)
                    + r * _RB) < _N
            om = jnp.where(mask, o, 0.0)

            @pl.when(r == 0)
            def _():
                so_ref[...] = jnp.zeros((8, fout), jnp.float32)

            so_ref[0:1, :] = so_ref[0:1, :] + jnp.sum(om, axis=0, keepdims=True)
            so_ref[1:2, :] = so_ref[1:2, :] + jnp.sum(om * om, axis=0,
                                                      keepdims=True)

    out_specs = [pl.BlockSpec((_RB, fout), lambda r: (r, 0)),
                 pl.BlockSpec((8, fout), lambda r: (0, 0))]
    out_shape = [jax.ShapeDtypeStruct((_NPAD, fout), jnp.float32),
                 jax.ShapeDtypeStruct((8, fout), jnp.float32)]
    res = pl.pallas_call(
        body,
        grid=(_R,),
        in_specs=[
            pl.BlockSpec((_RB, fin), lambda r: (r, 0)),
            pl.BlockSpec((8, fin), lambda r: (0, 0)),
            pl.BlockSpec((1, fin), lambda r: (0, 0)),
            pl.BlockSpec((1, fin), lambda r: (0, 0)),
            pl.BlockSpec((fin, fout), lambda r: (0, 0)),
            pl.BlockSpec((1, fout), lambda r: (0, 0)),
        ],
        out_specs=out_specs,
        out_shape=out_shape,
    )(h, s, g, be, w, b)
    return res[0], res[1]


def kernel(x, edge_index, W1, b1, W2, b2, g1, be1, Wl2, bl2, g2, be2,
           Wl3, bl3, g3, be3, Wl4, bl4):
    n = x.shape[0]
    e = edge_index.shape[1]
    blocks1 = -(-e // (_CH * 32 * 16))
    chunks = blocks1 * 32 * 16
    epad = chunks * _CH

    src = edge_index[0].astype(jnp.int32)
    dst = edge_index[1].astype(jnp.int32)
    fill = jnp.full((epad - e,), _NPAD - 1, jnp.int32)
    srcp = jnp.concatenate([src, fill])
    dstp = jnp.concatenate([dst, fill])

    x16 = jnp.pad(x, ((0, _NPAD - n), (0, 16 - x.shape[1])))
    w1p = jnp.pad(W1, ((0, 16 - W1.shape[0]), (0, 160 - W1.shape[1])))
    b1p = jnp.pad(b1, (0, 160 - b1.shape[0])).reshape(1, 160)
    w2p = jnp.pad(W2, ((0, 160 - W2.shape[0]), (0, 0)))
    b2r = b2.reshape(1, 200)
    wl4p = jnp.pad(Wl4, ((0, 0), (0, 8 - Wl4.shape[1])))
    bl4p = jnp.pad(bl4, (0, 8 - bl4.shape[0])).reshape(1, 8)

    ones = jnp.ones((_EB,), jnp.float32)
    z1 = jnp.zeros((_RPT,), jnp.float32)
    zrows = jnp.zeros((_RPT, 16), jnp.float32)

    degp = _make_deg(epad)(dstp, ones, z1)
    degp_t = degp.reshape(2, _NPAD).T
    y0, dv16 = _t1_call(degp_t, x16)

    out1p = _make_conv1(epad)(srcp, dstp, y0, zrows)
    out1p = out1p.reshape(2, _NPAD, 16)
    y1 = _t2_call(out1p, y0, dv16, w1p, b1p)

    out3 = _make_conv2(epad)(srcp, dstp, y1, zrows)
    out3 = out3.reshape(_NP2, _NPAD, 16)
    h2, s1 = _d1_call(out3, y1, dv16, w2p, b2r)

    h3, s2 = _bn_mlp_call(h2, s1, g1.reshape(1, 200), be1.reshape(1, 200),
                          Wl2, bl2.reshape(1, 400), 200, 400, False)
    h4, s3 = _bn_mlp_call(h3, s2, g2.reshape(1, 400), be2.reshape(1, 400),
                          Wl3, bl3.reshape(1, 200), 400, 200, False)
    h5, _ = _bn_mlp_call(h4, s3, g3.reshape(1, 200), be3.reshape(1, 200),
                         wl4p, bl4p, 200, 8, True)
    return h5[:n, 0:1]
